# baseline (speedup 1.0000x reference)
import functools

import jax
import jax.numpy as jnp
from jax import lax
from jax.experimental import pallas as pl
from jax.experimental.pallas import tpu as pltpu, tpu_sc as plsc

_D = 6144
_B = 8192
_NC = 2
_NS = 16
_NBW = 8
_BPW = _B // _NBW
_CHB = 16
_NCB = _BPW // _CHB
_NBUF = 2

_mesh = plsc.VectorSubcoreMesh(core_axis_name="c", subcore_axis_name="s")


@functools.partial(
    pl.kernel,
    out_type=jax.ShapeDtypeStruct((_B, _D), jnp.float32),
    mesh=_mesh,
    scratch_types=[
        pltpu.VMEM((_BPW,), jnp.int32),
        pltpu.MemorySpace.VMEM_SHARED((4, _NBUF, _CHB, _D), jnp.float32),
        [pltpu.SemaphoreType.DMA] * _NBUF,
        [pltpu.SemaphoreType.DMA] * _NBUF,
    ],
)
def _lookup(w_hbm, xb_hbm, out_hbm, idx_bv, buf_sh, gb, pb):
    sid = lax.axis_index("s")
    cid = lax.axis_index("c")

    @pl.when(sid < 4)
    def _():
        bw = sid * _NC + cid
        base = bw * _BPW
        pltpu.sync_copy(xb_hbm.at[bw], idx_bv)

        def start_gather(b, c):
            rows = idx_bv[pl.ds(c * _CHB, _CHB)]
            for j in range(_CHB):
                pltpu.async_copy(
                    w_hbm.at[pl.ds(rows[j], 1)],
                    buf_sh.at[sid, b, pl.ds(j, 1)],
                    gb[b],
                )

        def wait_gather(b):
            for j in range(_CHB):
                pltpu.make_async_copy(
                    w_hbm.at[pl.ds(0, 1)], buf_sh.at[sid, b, pl.ds(j, 1)], gb[b]
                ).wait()

        def start_write(b, c):
            pltpu.async_copy(
                buf_sh.at[sid, b],
                out_hbm.at[pl.ds(base + c * _CHB, _CHB)],
                pb[b],
            )

        def wait_write(b):
            pltpu.make_async_copy(
                buf_sh.at[sid, b], out_hbm.at[pl.ds(base, _CHB)], pb[b]
            ).wait()

        for b in range(_NBUF):
            start_gather(b, b)

        def body(g, carry):
            for b in range(_NBUF):
                c = g * _NBUF + b
                wait_gather(b)
                start_write(b, c)
                wait_write(b)
                start_gather(b, c + _NBUF)
            return carry

        lax.fori_loop(0, _NCB // _NBUF - 1, body, 0)

        for b in range(_NBUF):
            wait_gather(b)
            start_write(b, _NCB - _NBUF + b)
        for b in range(_NBUF):
            wait_write(b)


def kernel(x, W):
    flat = x.reshape(-1)
    out = _lookup(W, flat.reshape(_NBW, _BPW))
    return out.reshape(x.shape + (W.shape[1],))
